# Initial kernel scaffold; baseline (speedup 1.0000x reference)
import jax
import jax.numpy as jnp
from jax.experimental import pallas as pl


def kernel(x, table):
    raise NotImplementedError("write your pallas kernel here")


# speedup vs baseline: 1.4588x; 1.4588x over previous
import functools

import jax
import jax.numpy as jnp
from jax import lax
from jax.experimental import pallas as pl
from jax.experimental.pallas import tpu as pltpu
from jax.experimental.pallas import tpu_sc as plsc

_EMBED_DIM = 32
_NUM_CORES = 2
_NUM_SUBCORES = 16
_NW = _NUM_CORES * _NUM_SUBCORES
_CHUNK = 1024


def _embed_body(x_hbm, table_hbm, out_hbm, idx_v, rows_v, sem):
    wid = lax.axis_index("s") * _NUM_CORES + lax.axis_index("c")
    n_total = x_hbm.shape[0]
    per_w = n_total // _NW
    n_chunks = per_w // _CHUNK
    base = wid * per_w

    def body(i, carry):
        off = base + i * _CHUNK
        pltpu.sync_copy(x_hbm.at[pl.ds(off, _CHUNK)], idx_v)
        pltpu.async_copy(table_hbm.at[idx_v], rows_v, sem).wait()
        pltpu.sync_copy(rows_v, out_hbm.at[pl.ds(off, _CHUNK)])
        return carry

    lax.fori_loop(0, n_chunks, body, 0)


def _make_lookup(n_rows):
    mesh = plsc.VectorSubcoreMesh(core_axis_name="c", subcore_axis_name="s")
    return functools.partial(
        pl.kernel,
        mesh=mesh,
        out_type=jax.ShapeDtypeStruct((n_rows, _EMBED_DIM), jnp.float32),
        scratch_types=[
            pltpu.VMEM((_CHUNK,), jnp.int32),
            pltpu.VMEM((_CHUNK, _EMBED_DIM), jnp.float32),
            pltpu.SemaphoreType.DMA,
        ],
        compiler_params=pltpu.CompilerParams(use_tc_tiling_on_sc=False),
    )(_embed_body)


@jax.jit
def kernel(x, table):
    b, h = x.shape
    flat = x.reshape(b * h).astype(jnp.int32)
    out = _make_lookup(b * h)(flat, table)
    return out.reshape(b, h, _EMBED_DIM)


# speedup vs baseline: 1.4922x; 1.0229x over previous
import functools

import jax
import jax.numpy as jnp
from jax import lax
from jax.experimental import pallas as pl
from jax.experimental.pallas import tpu as pltpu
from jax.experimental.pallas import tpu_sc as plsc

_EMBED_DIM = 32
_NUM_CORES = 2
_NUM_SUBCORES = 16
_NW = _NUM_CORES * _NUM_SUBCORES
_CHUNK = 1600


def _embed_body(x_hbm, table_hbm, out_hbm, idx_v, rows_v, sem_idx, sem_g, sem_st):
    wid = lax.axis_index("s") * _NUM_CORES + lax.axis_index("c")
    n_total = x_hbm.shape[0]
    per_w = n_total // _NW
    n_chunks = per_w // _CHUNK
    base = wid * per_w

    idx_dma = [None, None]
    st_dma = [None, None]

    def start_idx(i):
        b = i % 2
        idx_dma[b] = pltpu.make_async_copy(
            x_hbm.at[pl.ds(base + i * _CHUNK, _CHUNK)], idx_v.at[b], sem_idx.at[b]
        )
        idx_dma[b].start()

    start_idx(0)
    for i in range(n_chunks):
        b = i % 2
        if i + 1 < n_chunks:
            start_idx(i + 1)
        idx_dma[b].wait()
        if i >= 2:
            st_dma[b].wait()
        g = pltpu.make_async_copy(table_hbm.at[idx_v.at[b]], rows_v.at[b], sem_g)
        g.start()
        g.wait()
        st_dma[b] = pltpu.make_async_copy(
            rows_v.at[b], out_hbm.at[pl.ds(base + i * _CHUNK, _CHUNK)], sem_st.at[b]
        )
        st_dma[b].start()
    st_dma[(n_chunks - 2) % 2].wait()
    st_dma[(n_chunks - 1) % 2].wait()


def _make_lookup(n_rows):
    mesh = plsc.VectorSubcoreMesh(core_axis_name="c", subcore_axis_name="s")
    return functools.partial(
        pl.kernel,
        mesh=mesh,
        out_type=jax.ShapeDtypeStruct((n_rows, _EMBED_DIM), jnp.float32),
        scratch_types=[
            pltpu.VMEM((2, _CHUNK), jnp.int32),
            pltpu.VMEM((2, _CHUNK, _EMBED_DIM), jnp.float32),
            pltpu.SemaphoreType.DMA((2,)),
            pltpu.SemaphoreType.DMA,
            pltpu.SemaphoreType.DMA((2,)),
        ],
        compiler_params=pltpu.CompilerParams(use_tc_tiling_on_sc=False),
    )(_embed_body)


@jax.jit
def kernel(x, table):
    b, h = x.shape
    flat = x.reshape(b * h).astype(jnp.int32)
    out = _make_lookup(b * h)(flat, table)
    return out.reshape(b, h, _EMBED_DIM)


# speedup vs baseline: 1.4963x; 1.0027x over previous
import functools

import jax
import jax.numpy as jnp
from jax import lax
from jax.experimental import pallas as pl
from jax.experimental.pallas import tpu as pltpu
from jax.experimental.pallas import tpu_sc as plsc
from jax.experimental.layout import Format as _Format, Layout as _Layout

_EMBED_DIM = 32
_NUM_CORES = 2
_NUM_SUBCORES = 16
_NW = _NUM_CORES * _NUM_SUBCORES
_CHUNK = 1600


def _embed_body(x_hbm, table_hbm, out_hbm, idx_v, rows_v, sem_idx, sem_g, sem_st):
    wid = lax.axis_index("s") * _NUM_CORES + lax.axis_index("c")
    n_total = x_hbm.shape[0]
    per_w = n_total // _NW
    n_chunks = per_w // _CHUNK
    base = wid * per_w

    idx_dma = [None, None]
    st_dma = [None, None]

    def start_idx(i):
        b = i % 2
        idx_dma[b] = pltpu.make_async_copy(
            x_hbm.at[pl.ds(base + i * _CHUNK, _CHUNK)], idx_v.at[b], sem_idx.at[b]
        )
        idx_dma[b].start()

    start_idx(0)
    for i in range(n_chunks):
        b = i % 2
        if i + 1 < n_chunks:
            start_idx(i + 1)
        idx_dma[b].wait()
        if i >= 2:
            st_dma[b].wait()
        g = pltpu.make_async_copy(table_hbm.at[idx_v.at[b]], rows_v.at[b], sem_g)
        g.start()
        g.wait()
        st_dma[b] = pltpu.make_async_copy(
            rows_v.at[b], out_hbm.at[pl.ds(base + i * _CHUNK, _CHUNK)], sem_st.at[b]
        )
        st_dma[b].start()
    st_dma[(n_chunks - 2) % 2].wait()
    st_dma[(n_chunks - 1) % 2].wait()


def _make_lookup(n_rows):
    mesh = plsc.VectorSubcoreMesh(core_axis_name="c", subcore_axis_name="s")
    return functools.partial(
        pl.kernel,
        mesh=mesh,
        out_type=jax.ShapeDtypeStruct((n_rows, _EMBED_DIM), jnp.float32),
        scratch_types=[
            pltpu.VMEM((2, _CHUNK), jnp.int32),
            pltpu.VMEM((2, _CHUNK, _EMBED_DIM), jnp.float32),
            pltpu.SemaphoreType.DMA((2,)),
            pltpu.SemaphoreType.DMA,
            pltpu.SemaphoreType.DMA((2,)),
        ],
        compiler_params=pltpu.CompilerParams(use_tc_tiling_on_sc=False),
    )(_embed_body)


def _kernel_impl(x, table):
    b, h = x.shape
    flat = x.reshape(b * h).astype(jnp.int32)
    out = _make_lookup(b * h)(flat, table)
    return out.reshape(b, h, _EMBED_DIM)


_kernel_impl.__name__ = "kernel"
_jitted = None


def kernel(x, table):
    global _jitted
    if _jitted is None:
        dev = jax.devices()[0]
        fmt = _Format(
            _Layout(major_to_minor=(0, 1, 2)),
            jax.sharding.SingleDeviceSharding(dev),
        )
        _jitted = jax.jit(_kernel_impl, out_shardings=fmt)
    return _jitted(x, table)
